# baseline (speedup 1.0000x reference)
import functools

import jax
import jax.numpy as jnp
from jax import lax
from jax.experimental import pallas as pl
from jax.experimental.pallas import tpu as pltpu
from jax.experimental.pallas import tpu_sc as plsc

NUM_EXPERTS = 64
TOP_K = 2
TOKENS = 16384
BT = 1024

_info = plsc.get_sparse_core_info()
_NC, _NS, _L = _info.num_cores, _info.num_subcores, _info.num_lanes
_NW = _NC * _NS
_TPW = TOKENS // _NW
_GROUPS = _TPW // _L


def _softmax_scores_kernel(x_ref, w_ref, p_ref):
    scores = lax.dot_general(
        w_ref[...], x_ref[...], (((1,), (1,)), ((), ())),
        preferred_element_type=jnp.float32,
    )
    m = jnp.max(scores, axis=0, keepdims=True)
    e = jnp.exp(scores - m)
    p_ref[...] = e / jnp.sum(e, axis=0, keepdims=True)


def _routing_body(p_hbm, bias_hbm, wout_hbm, iout_hbm,
                  p_v, bias_v, wout_v, iout_v):
    wid = lax.axis_index("s") * _NC + lax.axis_index("c")
    base = wid * _TPW
    pltpu.sync_copy(bias_hbm, bias_v)
    pltpu.sync_copy(p_hbm.at[:, pl.ds(base, _TPW)], p_v)

    def group(g, carry):
        t0 = g * _L
        neg = jnp.full((_L,), -jnp.inf, jnp.float32)
        zero_i = jnp.zeros((_L,), jnp.int32)
        zero_f = jnp.zeros((_L,), jnp.float32)
        bv, bi, bp = neg, zero_i, zero_f
        sv, si, sp = neg, zero_i, zero_f
        for e_i in range(NUM_EXPERTS):
            pv = p_v[e_i, pl.ds(t0, _L)]
            c = pv + bias_v[e_i, :]
            c1 = c > bv
            c2 = c > sv
            ei_vec = jnp.full((_L,), e_i, jnp.int32)
            sv = jnp.where(c1, bv, jnp.where(c2, c, sv))
            si = jnp.where(c1, bi, jnp.where(c2, ei_vec, si))
            sp = jnp.where(c1, bp, jnp.where(c2, pv, sp))
            bv = jnp.where(c1, c, bv)
            bi = jnp.where(c1, ei_vec, bi)
            bp = jnp.where(c1, pv, bp)
        wout_v[0, pl.ds(t0, _L)] = bp
        wout_v[1, pl.ds(t0, _L)] = sp
        iout_v[0, pl.ds(t0, _L)] = bi
        iout_v[1, pl.ds(t0, _L)] = si
        return carry

    lax.fori_loop(0, _GROUPS, group, 0)
    pltpu.sync_copy(wout_v, wout_hbm.at[:, pl.ds(base, _TPW)])
    pltpu.sync_copy(iout_v, iout_hbm.at[:, pl.ds(base, _TPW)])


@jax.jit
def kernel(x, weight, bias):
    tokens = x.shape[0]
    pT = pl.pallas_call(
        _softmax_scores_kernel,
        grid=(tokens // BT,),
        in_specs=[
            pl.BlockSpec((BT, x.shape[1]), lambda i: (i, 0)),
            pl.BlockSpec(weight.shape, lambda i: (0, 0)),
        ],
        out_specs=pl.BlockSpec((NUM_EXPERTS, BT), lambda i: (0, i)),
        out_shape=jax.ShapeDtypeStruct((NUM_EXPERTS, tokens), jnp.float32),
    )(x, weight)

    bias_b = jnp.broadcast_to(bias[:, None], (NUM_EXPERTS, _L))

    routing = functools.partial(
        pl.kernel,
        mesh=plsc.VectorSubcoreMesh(core_axis_name="c", subcore_axis_name="s"),
        out_type=[
            jax.ShapeDtypeStruct((TOP_K, tokens), jnp.float32),
            jax.ShapeDtypeStruct((TOP_K, tokens), jnp.int32),
        ],
        scratch_types=[
            pltpu.VMEM((NUM_EXPERTS, _TPW), jnp.float32),
            pltpu.VMEM((NUM_EXPERTS, _L), jnp.float32),
            pltpu.VMEM((TOP_K, _TPW), jnp.float32),
            pltpu.VMEM((TOP_K, _TPW), jnp.int32),
        ],
    )(_routing_body)
    wT, iT = routing(pT, bias_b)
    return wT.T, iT.T
